# baseline (speedup 1.0000x reference)
import jax
import jax.numpy as jnp
from jax.experimental import pallas as pl
from jax.experimental.pallas import tpu as pltpu
from jax.experimental.pallas import tpu_sc as plsc

_BLK = 8192
_L = 16


def kernel(x, pe_table):
    B, S, H = x.shape
    x1 = x.reshape(B * S * H)
    pe1 = pe_table.reshape(pe_table.shape[0] * H)
    n_pe_blocks = (S * H) // _BLK

    mesh = plsc.VectorSubcoreMesh(core_axis_name="c", subcore_axis_name="s")

    @pl.kernel(out_type=jax.ShapeDtypeStruct((B * S * H,), x.dtype), mesh=mesh)
    def pe_add_sc(x_hbm, pe_hbm, o_hbm):
        def body(x_vmem, pe_vmem, o_vmem):
            @plsc.parallel_loop(0, _BLK, step=_L, unroll=8)
            def _chunk(c):
                slc = pl.ds(c, _L)
                o_vmem.at[slc][...] = x_vmem.at[slc][...] + pe_vmem.at[slc][...]

        pltpu.emit_pipeline(
            body,
            grid=((B * S * H) // _BLK,),
            in_specs=[
                pl.BlockSpec((_BLK,), lambda i: (i,)),
                pl.BlockSpec((_BLK,), lambda i: (i % n_pe_blocks,)),
            ],
            out_specs=[pl.BlockSpec((_BLK,), lambda i: (i,))],
            core_axis_name=("c", "s"),
            dimension_semantics=(pltpu.PARALLEL,),
        )(x_hbm, pe_hbm, o_hbm)

    return pe_add_sc(x1, pe1).reshape(B, S, H)


# speedup vs baseline: 3.3349x; 3.3349x over previous
import jax
import jax.numpy as jnp
from jax.experimental import pallas as pl
from jax.experimental.pallas import tpu as pltpu
from jax.experimental.pallas import tpu_sc as plsc

_RB = 4
_L = 16


def kernel(x, pe_table):
    B, S, H = x.shape

    mesh = plsc.VectorSubcoreMesh(core_axis_name="c", subcore_axis_name="s")

    @pl.kernel(out_type=jax.ShapeDtypeStruct((B, S, H), x.dtype), mesh=mesh)
    def pe_add_sc(x_hbm, pe_hbm, o_hbm):
        def body(x_vmem, pe_vmem, o_vmem):
            @plsc.parallel_loop(0, _RB * H, step=_L, unroll=4)
            def _chunk(c):
                r = c // H
                col = c - r * H
                slc = pl.ds(col, _L)
                pe_chunk = pe_vmem.at[r].at[slc][...]
                for b in range(B):
                    o_vmem.at[b].at[r].at[slc][...] = (
                        x_vmem.at[b].at[r].at[slc][...] + pe_chunk
                    )

        pltpu.emit_pipeline(
            body,
            grid=(S // _RB,),
            in_specs=[
                pl.BlockSpec((B, _RB, H), lambda i: (0, i, 0)),
                pl.BlockSpec((_RB, H), lambda i: (i, 0)),
            ],
            out_specs=[pl.BlockSpec((B, _RB, H), lambda i: (0, i, 0))],
            core_axis_name=("c", "s"),
            dimension_semantics=(pltpu.PARALLEL,),
        )(x_hbm, pe_hbm, o_hbm)

    return pe_add_sc(x, pe_table)
